# baseline (speedup 1.0000x reference)
import functools

import jax
import jax.numpy as jnp
from jax import lax
from jax.experimental import pallas as pl
from jax.experimental.pallas import tpu as pltpu
from jax.experimental.pallas import tpu_sc as plsc

_B, _S, _D = 4, 8192, 1024
_NC, _NS = 2, 16
_NW = _NC * _NS
_ROWS_PER_W = _S // _NW
_SB = 16
_NSUB = _ROWS_PER_W // _SB
_BLK = _SB * _D
_NXB = 4
_UNROLL = 8


def _add_block(xb, pb):
    n = _BLK // (16 * _UNROLL)

    def body(j, carry):
        base = j * (16 * _UNROLL)
        for u in range(_UNROLL):
            sl = pl.ds(base + u * 16, 16)
            xb[sl] = xb[sl] + pb[sl]
        return carry

    lax.fori_loop(0, n, body, 0)


_LOOK = 2


def _sc_kernel_body(x_hbm, pos_hbm, out_hbm, xb0, xb1, xb2, xb3, pb,
                    ls0, ls1, ls2, ls3, ss0, ss1, ss2, ss3):
    xbufs = (xb0, xb1, xb2, xb3)
    load_sems = (ls0, ls1, ls2, ls3)
    store_sems = (ss0, ss1, ss2, ss3)
    wid = lax.axis_index("s") * _NC + lax.axis_index("c")
    row0 = wid * _ROWS_PER_W

    def x_slice(t):
        sub, b = divmod(t, _B)
        off = b * (_S * _D) + (row0 + sub * _SB) * _D
        return pl.ds(off, _BLK)

    nt = _NSUB * _B
    loads = [None] * nt
    stores = [None] * nt

    def start_load(t):
        k = t % _NXB
        loads[t] = pltpu.async_copy(x_hbm.at[x_slice(t)], xbufs[k], load_sems[k])

    for t in range(_LOOK):
        start_load(t)

    for t in range(nt):
        k = t % _NXB
        sub, b = divmod(t, _B)
        if b == 0:
            pltpu.sync_copy(pos_hbm.at[pl.ds((row0 + sub * _SB) * _D, _BLK)], pb)
        loads[t].wait()
        _add_block(xbufs[k], pb)
        stores[t] = pltpu.async_copy(xbufs[k], out_hbm.at[x_slice(t)],
                                     store_sems[k])
        nxt = t + _LOOK
        if nxt < nt:
            prev = nxt - _NXB
            if prev >= 0:
                stores[prev].wait()
            start_load(nxt)

    for t in range(max(0, nt - _NXB), nt):
        stores[t].wait()


def _kernel_sc(x, pos_table):
    mesh = plsc.VectorSubcoreMesh(core_axis_name="c", subcore_axis_name="s")
    run = functools.partial(
        pl.kernel,
        mesh=mesh,
        out_type=jax.ShapeDtypeStruct((_B * _S * _D,), jnp.float32),
        scratch_types=(
            [pltpu.VMEM((_BLK,), jnp.float32)] * (_NXB + 1)
            + [pltpu.SemaphoreType.DMA] * (2 * _NXB)
        ),
    )(_sc_kernel_body)
    out = run(x.reshape(-1), pos_table.reshape(-1))
    return out.reshape(_B, _S, _D)


def kernel(x, pos_table):
    return _kernel_sc(x, pos_table)


# speedup vs baseline: 2.7181x; 2.7181x over previous
import functools

import jax
import jax.numpy as jnp
from jax import lax
from jax.experimental import pallas as pl
from jax.experimental.pallas import tpu as pltpu
from jax.experimental.pallas import tpu_sc as plsc

_B, _S, _D = 4, 8192, 1024
_NC, _NS = 2, 16
_NW = _NC * _NS
_ROWS_PER_W = _S // _NW
_SB = 16
_NSUB = _ROWS_PER_W // _SB
_NXB = 4
_LOOK = 2
_UNROLL = 8


def _add_block(xb, pb):
    n_chunks_per_row = _D // 16

    def body2(j, carry):
        base = j * _UNROLL
        for u in range(_UNROLL):
            idx = base + u
            r = idx // n_chunks_per_row
            c = lax.rem(idx, n_chunks_per_row) * 16
            sl = pl.ds(c, 16)
            xb[r, sl] = xb[r, sl] + pb[r, sl]
        return carry

    lax.fori_loop(0, (_SB * n_chunks_per_row) // _UNROLL, body2, 0)


def _sc_kernel_body(x_hbm, pos_hbm, out_hbm, xb0, xb1, xb2, xb3, pb,
                    ls0, ls1, ls2, ls3, ss0, ss1, ss2, ss3):
    xbufs = (xb0, xb1, xb2, xb3)
    load_sems = (ls0, ls1, ls2, ls3)
    store_sems = (ss0, ss1, ss2, ss3)
    wid = lax.axis_index("s") * _NC + lax.axis_index("c")
    row0 = wid * _ROWS_PER_W

    def rows(t):
        sub, b = divmod(t, _B)
        return b, pl.ds(row0 + sub * _SB, _SB)

    nt = _NSUB * _B
    loads = [None] * nt
    stores = [None] * nt

    def start_load(t):
        k = t % _NXB
        b, sl = rows(t)
        loads[t] = pltpu.async_copy(x_hbm.at[b, sl], xbufs[k], load_sems[k])

    for t in range(_LOOK):
        start_load(t)

    for t in range(nt):
        k = t % _NXB
        sub, b = divmod(t, _B)
        if b == 0:
            pltpu.sync_copy(pos_hbm.at[pl.ds(row0 + sub * _SB, _SB)], pb)
        loads[t].wait()
        _add_block(xbufs[k], pb)
        bb, sl = rows(t)
        stores[t] = pltpu.async_copy(xbufs[k], out_hbm.at[bb, sl],
                                     store_sems[k])
        nxt = t + _LOOK
        if nxt < nt:
            prev = nxt - _NXB
            if prev >= 0:
                stores[prev].wait()
            start_load(nxt)

    for t in range(max(0, nt - _NXB), nt):
        stores[t].wait()


def _kernel_sc(x, pos_table):
    mesh = plsc.VectorSubcoreMesh(core_axis_name="c", subcore_axis_name="s")
    run = functools.partial(
        pl.kernel,
        mesh=mesh,
        out_type=jax.ShapeDtypeStruct((_B, _S, _D), jnp.float32),
        scratch_types=(
            [pltpu.VMEM((_SB, _D), jnp.float32)] * (_NXB + 1)
            + [pltpu.SemaphoreType.DMA] * (2 * _NXB)
        ),
        compiler_params=pltpu.CompilerParams(use_tc_tiling_on_sc=True),
    )(_sc_kernel_body)
    return run(x, pos_table)


def kernel(x, pos_table):
    return _kernel_sc(x, pos_table)
